# baseline (speedup 1.0000x reference)
import functools

import jax
import jax.numpy as jnp
from jax import lax
from jax.experimental import pallas as pl
from jax.experimental.pallas import tpu as pltpu
from jax.experimental.pallas import tpu_sc as plsc

N_NODES = 50000
N_PAD = 50176
E_EDGES = 800000
E_ROWS = 6400
E_PAD = E_ROWS * 128
LANES = 128
NC, NS = 2, 16
D = 64
DH = 32
ROWS_PER_TILE_AGG = E_ROWS // NS
ROWS_PER_WORKER_DEG = E_ROWS // (NC * NS)
NODES_PER_TILE = N_PAD // NS
RB = 512
GRID_R = N_PAD // RB
KBUF = 4
CH = 40
N_CHUNKS = ROWS_PER_TILE_AGG // CH


def _sc_degree_body(dst_hbm, deg_parts_hbm, dst_v, acc_v):
    c = lax.axis_index("c")
    s = lax.axis_index("s")
    w = c * NS + s
    ones16 = jnp.ones((16,), jnp.float32)
    zer = jnp.zeros((16,), jnp.float32)

    @pl.loop(0, N_PAD // 128)
    def _(i):
        for u in range(8):
            acc_v[pl.ds(i * 128 + u * 16, 16)] = zer

    row0 = w * ROWS_PER_WORKER_DEG
    pltpu.sync_copy(dst_hbm.at[pl.ds(row0, ROWS_PER_WORKER_DEG)], dst_v)

    @pl.loop(0, ROWS_PER_WORKER_DEG)
    def _(j):
        for u in range(8):
            idx = dst_v[j, pl.ds(u * 16, 16)]
            plsc.addupdate_scatter(acc_v, [idx], ones16)

    pltpu.sync_copy(acc_v, deg_parts_hbm.at[pl.ds(w * N_PAD, N_PAD)])


def _sc_degree(dst2):
    mesh = plsc.VectorSubcoreMesh(core_axis_name="c", subcore_axis_name="s")
    kern = pl.kernel(
        _sc_degree_body,
        out_type=jax.ShapeDtypeStruct((NC * NS * N_PAD,), jnp.float32),
        compiler_params=pltpu.CompilerParams(needs_layout_passes=False),
        mesh=mesh,
        scratch_types=[
            pltpu.VMEM((ROWS_PER_WORKER_DEG, LANES), jnp.int32),
            pltpu.VMEM((N_PAD,), jnp.float32),
        ],
    )
    return kern(dst2)


def _sc_agg_body(src_hbm, dst_hbm, table_hbm, acc_hbm,
                 src_v, dst_v, rows_v, acc_sh, sem_g, sem_s):
    c = lax.axis_index("c")
    s = lax.axis_index("s")

    zer = jnp.zeros((16,), jnp.float32)

    @pl.loop(0, LANES)
    def _(j):
        rows_v[0, j, pl.ds(0, 16)] = zer
        rows_v[0, j, pl.ds(16, 16)] = zer

    node0 = s * NODES_PER_TILE
    nfull = NODES_PER_TILE // 128

    @pl.loop(0, nfull)
    def _(i):
        pltpu.sync_copy(rows_v.at[0],
                        acc_sh.at[pl.ds(node0 + i * 128, 128)])

    pltpu.sync_copy(rows_v.at[0, pl.ds(0, NODES_PER_TILE - nfull * 128)],
                    acc_sh.at[pl.ds(node0 + nfull * 128,
                                    NODES_PER_TILE - nfull * 128)])

    plsc.subcore_barrier()

    row0 = s * ROWS_PER_TILE_AGG
    table_c = table_hbm.at[c]

    def g_start(j, b):
        pltpu.async_copy(table_c.at[src_v.at[j]], rows_v.at[b], sem_g.at[b])

    def g_wait(j, b):
        pltpu.make_async_copy(table_c.at[src_v.at[j]], rows_v.at[b],
                              sem_g.at[b]).wait()

    def s_start(j, b):
        pltpu.async_copy(rows_v.at[b], acc_sh.at[dst_v.at[j]], sem_s.at[b],
                         add=True)

    def s_wait(j, b):
        pltpu.make_async_copy(rows_v.at[b], acc_sh.at[dst_v.at[j]],
                              sem_s.at[b]).wait()

    @pl.loop(0, N_CHUNKS)
    def _(ch):
        r0 = row0 + ch * CH
        pltpu.sync_copy(src_hbm.at[pl.ds(r0, CH)], src_v)
        pltpu.sync_copy(dst_hbm.at[pl.ds(r0, CH)], dst_v)

        for b in range(KBUF):
            g_start(b, b)

        @pl.loop(0, CH // KBUF - 1)
        def _(g):
            r = g * KBUF
            for b in range(KBUF):
                g_wait(r + b, b)
                s_start(r + b, b)
            for b in range(KBUF):
                s_wait(r + b, b)
                g_start(r + KBUF + b, b)

        r_last = CH - KBUF
        for b in range(KBUF):
            g_wait(r_last + b, b)
            s_start(r_last + b, b)
        for b in range(KBUF):
            s_wait(r_last + b, b)

    plsc.subcore_barrier()

    pltpu.sync_copy(acc_sh.at[pl.ds(node0, NODES_PER_TILE)],
                    acc_hbm.at[c, pl.ds(node0, NODES_PER_TILE)])


def _sc_edge_aggregate(src2, dst2, table):
    mesh = plsc.VectorSubcoreMesh(core_axis_name="c", subcore_axis_name="s")
    kern = pl.kernel(
        _sc_agg_body,
        out_type=jax.ShapeDtypeStruct((NC, N_PAD, DH), jnp.float32),
        compiler_params=pltpu.CompilerParams(use_tc_tiling_on_sc=False),
        mesh=mesh,
        scratch_types=[
            pltpu.VMEM((CH, LANES), jnp.int32),
            pltpu.VMEM((CH, LANES), jnp.int32),
            pltpu.VMEM((KBUF, LANES, DH), jnp.float32),
            pltpu.VMEM_SHARED((N_PAD, DH), jnp.float32),
            pltpu.SemaphoreType.DMA((KBUF,)),
            pltpu.SemaphoreType.DMA((KBUF,)),
        ],
    )
    return kern(src2, dst2, table)


def _dis_block(parts_ref):
    ones = jnp.ones((NC * NS, 1), jnp.float32)
    deg = lax.dot_general(parts_ref[...], ones, (((0,), (0,)), ((), ())),
                          preferred_element_type=jnp.float32) + 1.0
    return lax.rsqrt(deg)


def _stage_b_kernel(parts_ref, z_ref, w1_ref, hcat_ref):
    dis = _dis_block(parts_ref)
    h = jnp.dot(z_ref[...], w1_ref[...],
                preferred_element_type=jnp.float32)
    hs = h * dis
    hcat_ref[0] = hs[:, :DH]
    hcat_ref[1] = hs[:, DH:]


def _stage_b(deg_parts, z, w1):
    return pl.pallas_call(
        _stage_b_kernel,
        grid=(GRID_R,),
        in_specs=[
            pl.BlockSpec((NC * NS, RB), lambda i: (0, i)),
            pl.BlockSpec((RB, D), lambda i: (i, 0)),
            pl.BlockSpec((D, D), lambda i: (0, 0)),
        ],
        out_specs=pl.BlockSpec((NC, RB, DH), lambda i: (0, i, 0)),
        out_shape=jax.ShapeDtypeStruct((NC, N_PAD, DH), jnp.float32),
    )(deg_parts, z, w1)


def _stage_c_kernel(parts_ref, acc_ref, hcat_ref, w2_ref, b1_ref, hcat2_ref):
    dis = _dis_block(parts_ref)
    x0 = jnp.maximum((acc_ref[0] + hcat_ref[0]) * dis + b1_ref[0], 0.0)
    x1 = jnp.maximum((acc_ref[1] + hcat_ref[1]) * dis + b1_ref[1], 0.0)
    w2 = w2_ref[...]
    h2 = (jnp.dot(x0, w2[:DH, :], preferred_element_type=jnp.float32)
          + jnp.dot(x1, w2[DH:, :], preferred_element_type=jnp.float32))
    hs = h2 * dis
    hcat2_ref[0] = hs[:, :DH]
    hcat2_ref[1] = hs[:, DH:]


def _stage_c(deg_parts, acc1, hcat1, w2, b1r):
    return pl.pallas_call(
        _stage_c_kernel,
        grid=(GRID_R,),
        in_specs=[
            pl.BlockSpec((NC * NS, RB), lambda i: (0, i)),
            pl.BlockSpec((NC, RB, DH), lambda i: (0, i, 0)),
            pl.BlockSpec((NC, RB, DH), lambda i: (0, i, 0)),
            pl.BlockSpec((D, D), lambda i: (0, 0)),
            pl.BlockSpec((NC, DH), lambda i: (0, 0)),
        ],
        out_specs=pl.BlockSpec((NC, RB, DH), lambda i: (0, i, 0)),
        out_shape=jax.ShapeDtypeStruct((NC, N_PAD, DH), jnp.float32),
    )(deg_parts, acc1, hcat1, w2, b1r)


def _stage_d_kernel(parts_ref, acc_ref, hcat_ref, b2_ref, out_ref):
    dis = _dis_block(parts_ref)
    y0 = jnp.maximum((acc_ref[0] + hcat_ref[0]) * dis + b2_ref[0], 0.0)
    y1 = jnp.maximum((acc_ref[1] + hcat_ref[1]) * dis + b2_ref[1], 0.0)
    out_ref[...] = jnp.concatenate([y0, y1], axis=1)


def _stage_d(deg_parts, acc2, hcat2, b2r):
    return pl.pallas_call(
        _stage_d_kernel,
        grid=(GRID_R,),
        in_specs=[
            pl.BlockSpec((NC * NS, RB), lambda i: (0, i)),
            pl.BlockSpec((NC, RB, DH), lambda i: (0, i, 0)),
            pl.BlockSpec((NC, RB, DH), lambda i: (0, i, 0)),
            pl.BlockSpec((NC, DH), lambda i: (0, 0)),
        ],
        out_specs=pl.BlockSpec((RB, D), lambda i: (i, 0)),
        out_shape=jax.ShapeDtypeStruct((N_PAD, D), jnp.float32),
    )(deg_parts, acc2, hcat2, b2r)


def kernel(z, edge_index, W1, b1, W2, b2):
    src = edge_index[0].astype(jnp.int32)
    dst = edge_index[1].astype(jnp.int32)
    pad = jnp.full((E_PAD - E_EDGES,), N_NODES, jnp.int32)
    src2 = jnp.concatenate([src, pad]).reshape(E_ROWS, LANES)
    dst2 = jnp.concatenate([dst, pad]).reshape(E_ROWS, LANES)

    b1r = b1.reshape(NC, DH)
    b2r = b2.reshape(NC, DH)

    deg_parts = _sc_degree(dst2).reshape(NC * NS, N_PAD)
    hcat1 = _stage_b(deg_parts, z, W1)
    acc1 = _sc_edge_aggregate(src2, dst2, hcat1)
    hcat2 = _stage_c(deg_parts, acc1, hcat1, W2, b1r)
    acc2 = _sc_edge_aggregate(src2, dst2, hcat2)
    out = _stage_d(deg_parts, acc2, hcat2, b2r)
    return out[:N_NODES]


# speedup vs baseline: 1.3069x; 1.3069x over previous
import functools

import jax
import jax.numpy as jnp
from jax import lax
from jax.experimental import pallas as pl
from jax.experimental.pallas import tpu as pltpu
from jax.experimental.pallas import tpu_sc as plsc

N_NODES = 50000
N_PAD = 50176
E_EDGES = 800000
E_ROWS = 6400
E_PAD = E_ROWS * 128
LANES = 128
NC, NS = 2, 16
D = 64
DH = 32
ROWS_PER_TILE_AGG = E_ROWS // NS
ROWS_PER_WORKER_DEG = E_ROWS // (NC * NS)
NODES_PER_TILE = N_PAD // NS
RB = 512
GRID_R = N_PAD // RB
KBUF = 8
CH = 40
N_CHUNKS = ROWS_PER_TILE_AGG // CH
EDT = jnp.bfloat16


def _sc_degree_body(dst_hbm, deg_parts_hbm, dst_v, acc_v):
    c = lax.axis_index("c")
    s = lax.axis_index("s")
    w = c * NS + s
    ones16 = jnp.ones((16,), jnp.float32)
    zer = jnp.zeros((16,), jnp.float32)

    @pl.loop(0, N_PAD // 128)
    def _(i):
        for u in range(8):
            acc_v[pl.ds(i * 128 + u * 16, 16)] = zer

    row0 = w * ROWS_PER_WORKER_DEG
    pltpu.sync_copy(dst_hbm.at[pl.ds(row0, ROWS_PER_WORKER_DEG)], dst_v)

    @pl.loop(0, ROWS_PER_WORKER_DEG)
    def _(j):
        for u in range(8):
            idx = dst_v[j, pl.ds(u * 16, 16)]
            plsc.addupdate_scatter(acc_v, [idx], ones16)

    pltpu.sync_copy(acc_v, deg_parts_hbm.at[pl.ds(w * N_PAD, N_PAD)])


def _sc_degree(dst2):
    mesh = plsc.VectorSubcoreMesh(core_axis_name="c", subcore_axis_name="s")
    kern = pl.kernel(
        _sc_degree_body,
        out_type=jax.ShapeDtypeStruct((NC * NS * N_PAD,), jnp.float32),
        compiler_params=pltpu.CompilerParams(needs_layout_passes=False),
        mesh=mesh,
        scratch_types=[
            pltpu.VMEM((ROWS_PER_WORKER_DEG, LANES), jnp.int32),
            pltpu.VMEM((N_PAD,), jnp.float32),
        ],
    )
    return kern(dst2)


def _sc_agg_body(src_hbm, dst_hbm, table_hbm, acc_hbm,
                 src_v, dst_v, rows_v, acc_sh, sem_g, sem_s):
    c = lax.axis_index("c")
    s = lax.axis_index("s")

    zer = jnp.zeros((DH,), EDT)

    @pl.loop(0, LANES)
    def _(j):
        rows_v[0, j, :] = zer

    node0 = s * NODES_PER_TILE
    nfull = NODES_PER_TILE // 128

    @pl.loop(0, nfull)
    def _(i):
        pltpu.sync_copy(rows_v.at[0],
                        acc_sh.at[pl.ds(node0 + i * 128, 128)])

    pltpu.sync_copy(rows_v.at[0, pl.ds(0, NODES_PER_TILE - nfull * 128)],
                    acc_sh.at[pl.ds(node0 + nfull * 128,
                                    NODES_PER_TILE - nfull * 128)])

    plsc.subcore_barrier()

    row0 = s * ROWS_PER_TILE_AGG
    table_c = table_hbm.at[c]

    def g_start(j, b):
        pltpu.async_copy(table_c.at[src_v.at[j]], rows_v.at[b], sem_g.at[b])

    def g_wait(j, b):
        pltpu.make_async_copy(table_c.at[src_v.at[j]], rows_v.at[b],
                              sem_g.at[b]).wait()

    def s_start(j, b):
        pltpu.async_copy(rows_v.at[b], acc_sh.at[dst_v.at[j]], sem_s.at[b],
                         add=True)

    def s_wait(j, b):
        pltpu.make_async_copy(rows_v.at[b], acc_sh.at[dst_v.at[j]],
                              sem_s.at[b]).wait()

    @pl.loop(0, N_CHUNKS)
    def _(ch):
        r0 = row0 + ch * CH
        pltpu.sync_copy(src_hbm.at[pl.ds(r0, CH)], src_v)
        pltpu.sync_copy(dst_hbm.at[pl.ds(r0, CH)], dst_v)

        for b in range(KBUF):
            g_start(b, b)

        @pl.loop(0, CH // KBUF - 1)
        def _(g):
            r = g * KBUF
            for b in range(KBUF):
                g_wait(r + b, b)
                s_start(r + b, b)
            for b in range(KBUF):
                s_wait(r + b, b)
                g_start(r + KBUF + b, b)

        r_last = CH - KBUF
        for b in range(KBUF):
            g_wait(r_last + b, b)
            s_start(r_last + b, b)
        for b in range(KBUF):
            s_wait(r_last + b, b)

    plsc.subcore_barrier()

    pltpu.sync_copy(acc_sh.at[pl.ds(node0, NODES_PER_TILE)],
                    acc_hbm.at[c, pl.ds(node0, NODES_PER_TILE)])


def _sc_edge_aggregate(src2, dst2, table):
    mesh = plsc.VectorSubcoreMesh(core_axis_name="c", subcore_axis_name="s")
    kern = pl.kernel(
        _sc_agg_body,
        out_type=jax.ShapeDtypeStruct((NC, N_PAD, DH), EDT),
        compiler_params=pltpu.CompilerParams(use_tc_tiling_on_sc=False),
        mesh=mesh,
        scratch_types=[
            pltpu.VMEM((CH, LANES), jnp.int32),
            pltpu.VMEM((CH, LANES), jnp.int32),
            pltpu.VMEM((KBUF, LANES, DH), EDT),
            pltpu.VMEM_SHARED((N_PAD, DH), EDT),
            pltpu.SemaphoreType.DMA((KBUF,)),
            pltpu.SemaphoreType.DMA((KBUF,)),
        ],
    )
    return kern(src2, dst2, table)


def _dis_block(parts_ref):
    ones = jnp.ones((NC * NS, 1), jnp.float32)
    deg = lax.dot_general(parts_ref[...], ones, (((0,), (0,)), ((), ())),
                          preferred_element_type=jnp.float32) + 1.0
    return lax.rsqrt(deg)


def _stage_b_kernel(parts_ref, z_ref, w1_ref, hcat_ref):
    dis = _dis_block(parts_ref)
    h = jnp.dot(z_ref[...], w1_ref[...],
                preferred_element_type=jnp.float32)
    hs = (h * dis).astype(EDT)
    hcat_ref[0] = hs[:, :DH]
    hcat_ref[1] = hs[:, DH:]


def _stage_b(deg_parts, z, w1):
    return pl.pallas_call(
        _stage_b_kernel,
        grid=(GRID_R,),
        in_specs=[
            pl.BlockSpec((NC * NS, RB), lambda i: (0, i)),
            pl.BlockSpec((RB, D), lambda i: (i, 0)),
            pl.BlockSpec((D, D), lambda i: (0, 0)),
        ],
        out_specs=pl.BlockSpec((NC, RB, DH), lambda i: (0, i, 0)),
        out_shape=jax.ShapeDtypeStruct((NC, N_PAD, DH), EDT),
    )(deg_parts, z, w1)


def _stage_c_kernel(parts_ref, acc_ref, hcat_ref, w2_ref, b1_ref, hcat2_ref):
    dis = _dis_block(parts_ref)
    acc0 = acc_ref[0].astype(jnp.float32)
    acc1 = acc_ref[1].astype(jnp.float32)
    h0 = hcat_ref[0].astype(jnp.float32)
    h1 = hcat_ref[1].astype(jnp.float32)
    x0 = jnp.maximum((acc0 + h0) * dis + b1_ref[0], 0.0)
    x1 = jnp.maximum((acc1 + h1) * dis + b1_ref[1], 0.0)
    w2 = w2_ref[...]
    h2 = (jnp.dot(x0, w2[:DH, :], preferred_element_type=jnp.float32)
          + jnp.dot(x1, w2[DH:, :], preferred_element_type=jnp.float32))
    hs = (h2 * dis).astype(EDT)
    hcat2_ref[0] = hs[:, :DH]
    hcat2_ref[1] = hs[:, DH:]


def _stage_c(deg_parts, acc1, hcat1, w2, b1r):
    return pl.pallas_call(
        _stage_c_kernel,
        grid=(GRID_R,),
        in_specs=[
            pl.BlockSpec((NC * NS, RB), lambda i: (0, i)),
            pl.BlockSpec((NC, RB, DH), lambda i: (0, i, 0)),
            pl.BlockSpec((NC, RB, DH), lambda i: (0, i, 0)),
            pl.BlockSpec((D, D), lambda i: (0, 0)),
            pl.BlockSpec((NC, DH), lambda i: (0, 0)),
        ],
        out_specs=pl.BlockSpec((NC, RB, DH), lambda i: (0, i, 0)),
        out_shape=jax.ShapeDtypeStruct((NC, N_PAD, DH), EDT),
    )(deg_parts, acc1, hcat1, w2, b1r)


def _stage_d_kernel(parts_ref, acc_ref, hcat_ref, b2_ref, out_ref):
    dis = _dis_block(parts_ref)
    acc0 = acc_ref[0].astype(jnp.float32)
    acc1 = acc_ref[1].astype(jnp.float32)
    h0 = hcat_ref[0].astype(jnp.float32)
    h1 = hcat_ref[1].astype(jnp.float32)
    y0 = jnp.maximum((acc0 + h0) * dis + b2_ref[0], 0.0)
    y1 = jnp.maximum((acc1 + h1) * dis + b2_ref[1], 0.0)
    out_ref[...] = jnp.concatenate([y0, y1], axis=1)


def _stage_d(deg_parts, acc2, hcat2, b2r):
    return pl.pallas_call(
        _stage_d_kernel,
        grid=(GRID_R,),
        in_specs=[
            pl.BlockSpec((NC * NS, RB), lambda i: (0, i)),
            pl.BlockSpec((NC, RB, DH), lambda i: (0, i, 0)),
            pl.BlockSpec((NC, RB, DH), lambda i: (0, i, 0)),
            pl.BlockSpec((NC, DH), lambda i: (0, 0)),
        ],
        out_specs=pl.BlockSpec((RB, D), lambda i: (i, 0)),
        out_shape=jax.ShapeDtypeStruct((N_PAD, D), jnp.float32),
    )(deg_parts, acc2, hcat2, b2r)


def kernel(z, edge_index, W1, b1, W2, b2):
    src = edge_index[0].astype(jnp.int32)
    dst = edge_index[1].astype(jnp.int32)
    pad = jnp.full((E_PAD - E_EDGES,), N_NODES, jnp.int32)
    src2 = jnp.concatenate([src, pad]).reshape(E_ROWS, LANES)
    dst2 = jnp.concatenate([dst, pad]).reshape(E_ROWS, LANES)

    b1r = b1.reshape(NC, DH)
    b2r = b2.reshape(NC, DH)

    deg_parts = _sc_degree(dst2).reshape(NC * NS, N_PAD)
    hcat1 = _stage_b(deg_parts, z, W1)
    acc1 = _sc_edge_aggregate(src2, dst2, hcat1)
    hcat2 = _stage_c(deg_parts, acc1, hcat1, W2, b1r)
    acc2 = _sc_edge_aggregate(src2, dst2, hcat2)
    out = _stage_d(deg_parts, acc2, hcat2, b2r)
    return out[:N_NODES]


# speedup vs baseline: 1.4096x; 1.0786x over previous
import jax
import jax.numpy as jnp
import numpy as np
from jax import lax
from jax.experimental import pallas as pl
from jax.experimental.pallas import tpu as pltpu
from jax.experimental.pallas import tpu_sc as plsc

N_NODES = 50000
N_PAD = 50176
E_EDGES = 800000
E_ROWS = 6400
E_PAD = E_ROWS * 128
LANES = 128
NC, NS = 2, 16
D = 64
DH = 32
ROWS_PER_TILE_AGG = E_ROWS // NS
ROWS_PER_WORKER_DEG = E_ROWS // (NC * NS)
NODES_PER_TILE = N_PAD // NS
RB = 512
GRID_R = N_PAD // RB
KBUF = 8
CH = 40
N_CHUNKS = ROWS_PER_TILE_AGG // CH
EDT = jnp.bfloat16


def _sc_degree_body(dst_hbm, deg_parts_hbm, dst_v, acc_v):
    c = lax.axis_index("c")
    s = lax.axis_index("s")
    w = c * NS + s
    ones16 = jnp.ones((16,), jnp.float32)
    zer = jnp.zeros((16,), jnp.float32)

    @pl.loop(0, N_PAD // 128)
    def _(i):
        for u in range(8):
            acc_v[pl.ds(i * 128 + u * 16, 16)] = zer

    row0 = w * ROWS_PER_WORKER_DEG
    pltpu.sync_copy(dst_hbm.at[pl.ds(row0, ROWS_PER_WORKER_DEG)], dst_v)

    @pl.loop(0, ROWS_PER_WORKER_DEG)
    def _(j):
        for u in range(8):
            idx = dst_v[j, pl.ds(u * 16, 16)]
            plsc.addupdate_scatter(acc_v, [idx], ones16)

    pltpu.sync_copy(acc_v, deg_parts_hbm.at[pl.ds(w * N_PAD, N_PAD)])


def _sc_degree(dst2):
    mesh = plsc.VectorSubcoreMesh(core_axis_name="c", subcore_axis_name="s")
    kern = pl.kernel(
        _sc_degree_body,
        out_type=jax.ShapeDtypeStruct((NC * NS * N_PAD,), jnp.float32),
        compiler_params=pltpu.CompilerParams(needs_layout_passes=False),
        mesh=mesh,
        scratch_types=[
            pltpu.VMEM((ROWS_PER_WORKER_DEG, LANES), jnp.int32),
            pltpu.VMEM((N_PAD,), jnp.float32),
        ],
    )
    return kern(dst2)


def _sc_agg_body(src_hbm, dst_hbm, table_hbm, acc_hbm,
                 src_v, dst_v, rows_v, acc_sh, sem_g, sem_s):
    c = lax.axis_index("c")
    s = lax.axis_index("s")

    zer = jnp.zeros((DH,), EDT)

    @pl.loop(0, LANES)
    def _(j):
        rows_v[0, j, :] = zer

    node0 = s * NODES_PER_TILE
    nfull = NODES_PER_TILE // 128

    @pl.loop(0, nfull)
    def _(i):
        pltpu.sync_copy(rows_v.at[0],
                        acc_sh.at[pl.ds(node0 + i * 128, 128)])

    pltpu.sync_copy(rows_v.at[0, pl.ds(0, NODES_PER_TILE - nfull * 128)],
                    acc_sh.at[pl.ds(node0 + nfull * 128,
                                    NODES_PER_TILE - nfull * 128)])

    plsc.subcore_barrier()

    row0 = s * ROWS_PER_TILE_AGG
    table_c = table_hbm.at[c]

    def g_start(j, b):
        pltpu.async_copy(table_c.at[src_v.at[j]], rows_v.at[b], sem_g.at[b])

    def g_wait(j, b):
        pltpu.make_async_copy(table_c.at[src_v.at[j]], rows_v.at[b],
                              sem_g.at[b]).wait()

    def s_start(j, b):
        pltpu.async_copy(rows_v.at[b], acc_sh.at[dst_v.at[j]], sem_s.at[b],
                         add=True)

    def s_wait(j, b):
        pltpu.make_async_copy(rows_v.at[b], acc_sh.at[dst_v.at[j]],
                              sem_s.at[b]).wait()

    @pl.loop(0, N_CHUNKS)
    def _(ch):
        r0 = row0 + ch * CH
        pltpu.sync_copy(src_hbm.at[pl.ds(r0, CH)], src_v)
        pltpu.sync_copy(dst_hbm.at[pl.ds(r0, CH)], dst_v)

        for b in range(KBUF):
            g_start(b, b)

        @pl.loop(0, CH // KBUF - 1)
        def _(g):
            r = g * KBUF
            for b in range(KBUF):
                g_wait(r + b, b)
                s_start(r + b, b)
            for b in range(KBUF):
                s_wait(r + b, b)
                g_start(r + KBUF + b, b)

        r_last = CH - KBUF
        for b in range(KBUF):
            g_wait(r_last + b, b)
            s_start(r_last + b, b)
        for b in range(KBUF):
            s_wait(r_last + b, b)

    plsc.subcore_barrier()

    pltpu.sync_copy(acc_sh.at[pl.ds(node0, NODES_PER_TILE)],
                    acc_hbm.at[c, pl.ds(node0, NODES_PER_TILE)])


def _sc_edge_aggregate(src2, dst2, table):
    mesh = plsc.VectorSubcoreMesh(core_axis_name="c", subcore_axis_name="s")
    kern = pl.kernel(
        _sc_agg_body,
        out_type=jax.ShapeDtypeStruct((NC, N_PAD, DH), EDT),
        compiler_params=pltpu.CompilerParams(use_tc_tiling_on_sc=False),
        mesh=mesh,
        scratch_types=[
            pltpu.VMEM((CH, LANES), jnp.int32),
            pltpu.VMEM((CH, LANES), jnp.int32),
            pltpu.VMEM((KBUF, LANES, DH), EDT),
            pltpu.VMEM_SHARED((N_PAD, DH), EDT),
            pltpu.SemaphoreType.DMA((KBUF,)),
            pltpu.SemaphoreType.DMA((KBUF,)),
        ],
    )
    return kern(src2, dst2, table)


def _dis_block(parts_ref):
    ones = jnp.ones((NC * NS, 1), jnp.float32)
    deg = lax.dot_general(parts_ref[...], ones, (((0,), (0,)), ((), ())),
                          preferred_element_type=jnp.float32) + 1.0
    return lax.rsqrt(deg)


RP = RB // 8
NPP = N_PAD // 8


def _pack_mats(w):
    q = lax.broadcasted_iota(jnp.int32, (RP, RB), 0)
    r0 = lax.broadcasted_iota(jnp.int32, (RP, RB), 1)
    t2 = (r0 // 8 == q).astype(jnp.float32)
    r1 = lax.broadcasted_iota(jnp.int32, (RB, 8 * w), 0)
    l1 = lax.broadcasted_iota(jnp.int32, (RB, 8 * w), 1)
    m = (r1 % 8 == l1 // w).astype(jnp.float32)
    return t2, m


def _pack8(x, w):
    t2, m = _pack_mats(w)
    tiled = jnp.concatenate([x] * 8, axis=1)
    return lax.dot_general(t2, tiled * m, (((1,), (0,)), ((), ())),
                           preferred_element_type=jnp.float32)


def _dis_packed(parts_ref):
    dis = _dis_block(parts_ref)
    return _pack8(jnp.broadcast_to(dis, (RB, DH)), DH)


def _stage_b_kernel(parts_ref, z_ref, w1_ref, hcat_ref):
    dis = _dis_block(parts_ref)
    h = jnp.dot(z_ref[...], w1_ref[...],
                preferred_element_type=jnp.float32)
    hs = h * dis
    hcat_ref[0] = _pack8(hs[:, :DH], DH).astype(EDT)
    hcat_ref[1] = _pack8(hs[:, DH:], DH).astype(EDT)


def _stage_b(deg_parts, z, w1):
    return pl.pallas_call(
        _stage_b_kernel,
        grid=(GRID_R,),
        in_specs=[
            pl.BlockSpec((NC * NS, RB), lambda i: (0, i)),
            pl.BlockSpec((RB, D), lambda i: (i, 0)),
            pl.BlockSpec((D, D), lambda i: (0, 0)),
        ],
        out_specs=pl.BlockSpec((NC, RP, 256), lambda i: (0, i, 0)),
        out_shape=jax.ShapeDtypeStruct((NC, NPP, 256), EDT),
    )(deg_parts, z, w1)


def _stage_c_kernel(parts_ref, acc_ref, hcat_ref, w2k_ref, b1_ref, hcat2_ref):
    dis_p = _dis_packed(parts_ref)
    x_p = []
    for c in range(NC):
        a = acc_ref[c].astype(jnp.float32)
        h = hcat_ref[c].astype(jnp.float32)
        x_p.append(jnp.maximum((a + h) * dis_p + b1_ref[c], 0.0))
    for c in range(NC):
        h2 = (jnp.dot(x_p[0], w2k_ref[0, c],
                      preferred_element_type=jnp.float32)
              + jnp.dot(x_p[1], w2k_ref[1, c],
                        preferred_element_type=jnp.float32))
        hcat2_ref[c] = (h2 * dis_p).astype(EDT)


def _stage_c(deg_parts, acc1, hcat1, w2k, b1p):
    return pl.pallas_call(
        _stage_c_kernel,
        grid=(GRID_R,),
        in_specs=[
            pl.BlockSpec((NC * NS, RB), lambda i: (0, i)),
            pl.BlockSpec((NC, RP, 256), lambda i: (0, i, 0)),
            pl.BlockSpec((NC, RP, 256), lambda i: (0, i, 0)),
            pl.BlockSpec((NC, NC, 256, 256), lambda i: (0, 0, 0, 0)),
            pl.BlockSpec((NC, 256), lambda i: (0, 0)),
        ],
        out_specs=pl.BlockSpec((NC, RP, 256), lambda i: (0, i, 0)),
        out_shape=jax.ShapeDtypeStruct((NC, NPP, 256), EDT),
    )(deg_parts, acc1, hcat1, w2k, b1p)


def _stage_d_kernel(parts_ref, acc_ref, hcat_ref, pa_ref, b2_ref, out_ref):
    dis_p = _dis_packed(parts_ref)
    y_p = []
    for c in range(NC):
        a = acc_ref[c].astype(jnp.float32)
        h = hcat_ref[c].astype(jnp.float32)
        y_p.append(jnp.maximum((a + h) * dis_p + b2_ref[c], 0.0))
    out_ref[...] = (
        jnp.dot(y_p[0], pa_ref[0], preferred_element_type=jnp.float32)
        + jnp.dot(y_p[1], pa_ref[1], preferred_element_type=jnp.float32))


def _stage_d(deg_parts, acc2, hcat2, pa, b2p):
    return pl.pallas_call(
        _stage_d_kernel,
        grid=(GRID_R,),
        in_specs=[
            pl.BlockSpec((NC * NS, RB), lambda i: (0, i)),
            pl.BlockSpec((NC, RP, 256), lambda i: (0, i, 0)),
            pl.BlockSpec((NC, RP, 256), lambda i: (0, i, 0)),
            pl.BlockSpec((NC, 256, 512), lambda i: (0, 0, 0)),
            pl.BlockSpec((NC, 256), lambda i: (0, 0)),
        ],
        out_specs=pl.BlockSpec((RP, 512), lambda i: (i, 0)),
        out_shape=jax.ShapeDtypeStruct((N_NODES * D // 512, 512), jnp.float32),
    )(deg_parts, acc2, hcat2, pa, b2p)


def kernel(z, edge_index, W1, b1, W2, b2):
    src = edge_index[0].astype(jnp.int32)
    dst = edge_index[1].astype(jnp.int32)
    pad = jnp.full((E_PAD - E_EDGES,), N_NODES, jnp.int32)
    src2 = jnp.concatenate([src, pad]).reshape(E_ROWS, LANES)
    dst2 = jnp.concatenate([dst, pad]).reshape(E_ROWS, LANES)

    eye8 = jnp.eye(8, dtype=jnp.float32)
    w2k = jnp.stack([
        jnp.stack([jnp.kron(eye8, W2[32 * h:32 * h + 32, 32 * c:32 * c + 32])
                   for c in range(NC)])
        for h in range(NC)])
    b1p = jnp.stack([jnp.tile(b1[32 * c:32 * c + 32], 8) for c in range(NC)])
    b2p = jnp.stack([jnp.tile(b2[32 * c:32 * c + 32], 8) for c in range(NC)])
    i32 = np.arange(256)
    j8, k32 = i32 // 32, i32 % 32
    pa_np = np.zeros((NC, 256, 512), np.float32)
    for h in range(NC):
        pa_np[h, i32, 64 * j8 + 32 * h + k32] = 1.0
    pa = jnp.asarray(pa_np)

    deg_parts = _sc_degree(dst2).reshape(NC * NS, N_PAD)
    hcat1 = _stage_b(deg_parts, z, W1)
    acc1 = _sc_edge_aggregate(src2, dst2, hcat1.reshape(NC, N_PAD, DH))
    hcat2 = _stage_c(deg_parts, acc1.reshape(NC, NPP, 256), hcat1, w2k, b1p)
    acc2 = _sc_edge_aggregate(src2, dst2, hcat2.reshape(NC, N_PAD, DH))
    out = _stage_d(deg_parts, acc2.reshape(NC, NPP, 256), hcat2, pa, b2p)
    return out.reshape(N_NODES, D)


# speedup vs baseline: 1.4940x; 1.0599x over previous
import jax
import jax.numpy as jnp
import numpy as np
from jax import lax
from jax.experimental import pallas as pl
from jax.experimental.pallas import tpu as pltpu
from jax.experimental.pallas import tpu_sc as plsc

N_NODES = 50000
N_PAD = 50176
E_EDGES = 800000
E_ROWS = 6400
E_PAD = E_ROWS * 128
LANES = 128
NC, NS = 2, 16
D = 64
DH = 32
ROWS_PER_TILE_AGG = E_ROWS // NS
ROWS_PER_WORKER_DEG = E_ROWS // (NC * NS)
NODES_PER_TILE = N_PAD // NS
RB = 512
GRID_R = N_PAD // RB
KBUF = 10
CH = 50
N_CHUNKS = ROWS_PER_TILE_AGG // CH
EDT = jnp.bfloat16


def _sc_degree_body(dst_hbm, deg_parts_hbm, dst_v, acc_v):
    c = lax.axis_index("c")
    s = lax.axis_index("s")
    w = c * NS + s
    ones16 = jnp.ones((16,), jnp.float32)
    zer = jnp.zeros((16,), jnp.float32)

    @pl.loop(0, N_PAD // 128)
    def _(i):
        for u in range(8):
            acc_v[pl.ds(i * 128 + u * 16, 16)] = zer

    row0 = w * ROWS_PER_WORKER_DEG
    pltpu.sync_copy(dst_hbm.at[pl.ds(row0, ROWS_PER_WORKER_DEG)], dst_v)

    @pl.loop(0, ROWS_PER_WORKER_DEG)
    def _(j):
        for u in range(8):
            idx = dst_v[j, pl.ds(u * 16, 16)]
            plsc.addupdate_scatter(acc_v, [idx], ones16)

    pltpu.sync_copy(acc_v, deg_parts_hbm.at[pl.ds(w * N_PAD, N_PAD)])


def _sc_degree(dst2):
    mesh = plsc.VectorSubcoreMesh(core_axis_name="c", subcore_axis_name="s")
    kern = pl.kernel(
        _sc_degree_body,
        out_type=jax.ShapeDtypeStruct((NC * NS * N_PAD,), jnp.float32),
        compiler_params=pltpu.CompilerParams(needs_layout_passes=False),
        mesh=mesh,
        scratch_types=[
            pltpu.VMEM((ROWS_PER_WORKER_DEG, LANES), jnp.int32),
            pltpu.VMEM((N_PAD,), jnp.float32),
        ],
    )
    return kern(dst2)


def _sc_agg_body(src_hbm, dst_hbm, table_hbm, acc_hbm,
                 src_v, dst_v, rows_v, acc_sh, sem_g, sem_s):
    c = lax.axis_index("c")
    s = lax.axis_index("s")

    zer = jnp.zeros((DH,), EDT)

    @pl.loop(0, LANES)
    def _(j):
        rows_v[0, j, :] = zer

    node0 = s * NODES_PER_TILE
    nfull = NODES_PER_TILE // 128

    @pl.loop(0, nfull)
    def _(i):
        pltpu.sync_copy(rows_v.at[0],
                        acc_sh.at[pl.ds(node0 + i * 128, 128)])

    pltpu.sync_copy(rows_v.at[0, pl.ds(0, NODES_PER_TILE - nfull * 128)],
                    acc_sh.at[pl.ds(node0 + nfull * 128,
                                    NODES_PER_TILE - nfull * 128)])

    plsc.subcore_barrier()

    row0 = s * ROWS_PER_TILE_AGG
    table_c = table_hbm.at[c]

    def g_start(j, b):
        pltpu.async_copy(table_c.at[src_v.at[j]], rows_v.at[b], sem_g.at[b])

    def g_wait(j, b):
        pltpu.make_async_copy(table_c.at[src_v.at[j]], rows_v.at[b],
                              sem_g.at[b]).wait()

    def s_start(j, b):
        pltpu.async_copy(rows_v.at[b], acc_sh.at[dst_v.at[j]], sem_s.at[b],
                         add=True)

    def s_wait(j, b):
        pltpu.make_async_copy(rows_v.at[b], acc_sh.at[dst_v.at[j]],
                              sem_s.at[b]).wait()

    @pl.loop(0, N_CHUNKS)
    def _(ch):
        r0 = row0 + ch * CH
        pltpu.sync_copy(src_hbm.at[pl.ds(r0, CH)], src_v)
        pltpu.sync_copy(dst_hbm.at[pl.ds(r0, CH)], dst_v)

        for b in range(KBUF):
            g_start(b, b)

        @pl.loop(0, CH // KBUF - 1)
        def _(g):
            r = g * KBUF
            for b in range(KBUF):
                g_wait(r + b, b)
                s_start(r + b, b)
            for b in range(KBUF):
                s_wait(r + b, b)
                g_start(r + KBUF + b, b)

        r_last = CH - KBUF
        for b in range(KBUF):
            g_wait(r_last + b, b)
            s_start(r_last + b, b)
        for b in range(KBUF):
            s_wait(r_last + b, b)

    plsc.subcore_barrier()

    pltpu.sync_copy(acc_sh.at[pl.ds(node0, NODES_PER_TILE)],
                    acc_hbm.at[c, pl.ds(node0, NODES_PER_TILE)])


def _sc_edge_aggregate(src2, dst2, table):
    mesh = plsc.VectorSubcoreMesh(core_axis_name="c", subcore_axis_name="s")
    kern = pl.kernel(
        _sc_agg_body,
        out_type=jax.ShapeDtypeStruct((NC, N_PAD, DH), EDT),
        compiler_params=pltpu.CompilerParams(use_tc_tiling_on_sc=False),
        mesh=mesh,
        scratch_types=[
            pltpu.VMEM((CH, LANES), jnp.int32),
            pltpu.VMEM((CH, LANES), jnp.int32),
            pltpu.VMEM((KBUF, LANES, DH), EDT),
            pltpu.VMEM_SHARED((N_PAD, DH), EDT),
            pltpu.SemaphoreType.DMA((KBUF,)),
            pltpu.SemaphoreType.DMA((KBUF,)),
        ],
    )
    return kern(src2, dst2, table)


def _dis_block(parts_ref):
    ones = jnp.ones((NC * NS, 1), jnp.float32)
    deg = lax.dot_general(parts_ref[...], ones, (((0,), (0,)), ((), ())),
                          preferred_element_type=jnp.float32) + 1.0
    return lax.rsqrt(deg)


RP = RB // 8
NPP = N_PAD // 8


def _pack_mats(w):
    q = lax.broadcasted_iota(jnp.int32, (RP, RB), 0)
    r0 = lax.broadcasted_iota(jnp.int32, (RP, RB), 1)
    t2 = (r0 // 8 == q).astype(jnp.float32)
    r1 = lax.broadcasted_iota(jnp.int32, (RB, 8 * w), 0)
    l1 = lax.broadcasted_iota(jnp.int32, (RB, 8 * w), 1)
    m = (r1 % 8 == l1 // w).astype(jnp.float32)
    return t2, m


def _pack8(x, w):
    t2, m = _pack_mats(w)
    tiled = jnp.concatenate([x] * 8, axis=1)
    return lax.dot_general(t2, tiled * m, (((1,), (0,)), ((), ())),
                           preferred_element_type=jnp.float32)


def _dis_packed(parts_ref):
    dis = _dis_block(parts_ref)
    return _pack8(jnp.broadcast_to(dis, (RB, DH)), DH)


def _stage_b_kernel(parts_ref, zp_ref, w1k_ref, hcat_ref, disp_ref):
    dis_p = _dis_packed(parts_ref)
    for c in range(NC):
        hp = jnp.dot(zp_ref[...], w1k_ref[c],
                     preferred_element_type=jnp.float32)
        hcat_ref[c] = (hp * dis_p).astype(EDT)
    disp_ref[...] = dis_p


def _stage_b(deg_parts, z_p, w1k):
    return pl.pallas_call(
        _stage_b_kernel,
        grid=(GRID_R,),
        in_specs=[
            pl.BlockSpec((NC * NS, RB), lambda i: (0, i)),
            pl.BlockSpec((RP, 512), lambda i: (i, 0)),
            pl.BlockSpec((NC, 512, 256), lambda i: (0, 0, 0)),
        ],
        out_specs=[
            pl.BlockSpec((NC, RP, 256), lambda i: (0, i, 0)),
            pl.BlockSpec((RP, 256), lambda i: (i, 0)),
        ],
        out_shape=[
            jax.ShapeDtypeStruct((NC, NPP, 256), EDT),
            jax.ShapeDtypeStruct((NPP, 256), jnp.float32),
        ],
    )(deg_parts, z_p, w1k)


def _stage_c_kernel(disp_ref, acc_ref, hcat_ref, w2k_ref, b1_ref, hcat2_ref):
    dis_p = disp_ref[...]
    x_p = []
    for c in range(NC):
        a = acc_ref[c].astype(jnp.float32)
        h = hcat_ref[c].astype(jnp.float32)
        x_p.append(jnp.maximum((a + h) * dis_p + b1_ref[c], 0.0))
    for c in range(NC):
        h2 = (jnp.dot(x_p[0], w2k_ref[0, c],
                      preferred_element_type=jnp.float32)
              + jnp.dot(x_p[1], w2k_ref[1, c],
                        preferred_element_type=jnp.float32))
        hcat2_ref[c] = (h2 * dis_p).astype(EDT)


def _stage_c(dis_pk, acc1, hcat1, w2k, b1p):
    return pl.pallas_call(
        _stage_c_kernel,
        grid=(GRID_R,),
        in_specs=[
            pl.BlockSpec((RP, 256), lambda i: (i, 0)),
            pl.BlockSpec((NC, RP, 256), lambda i: (0, i, 0)),
            pl.BlockSpec((NC, RP, 256), lambda i: (0, i, 0)),
            pl.BlockSpec((NC, NC, 256, 256), lambda i: (0, 0, 0, 0)),
            pl.BlockSpec((NC, 256), lambda i: (0, 0)),
        ],
        out_specs=pl.BlockSpec((NC, RP, 256), lambda i: (0, i, 0)),
        out_shape=jax.ShapeDtypeStruct((NC, NPP, 256), EDT),
    )(dis_pk, acc1, hcat1, w2k, b1p)


def _stage_d_kernel(disp_ref, acc_ref, hcat_ref, pa_ref, b2_ref, out_ref):
    dis_p = disp_ref[...]
    y_p = []
    for c in range(NC):
        a = acc_ref[c].astype(jnp.float32)
        h = hcat_ref[c].astype(jnp.float32)
        y_p.append(jnp.maximum((a + h) * dis_p + b2_ref[c], 0.0))
    out_ref[...] = (
        jnp.dot(y_p[0], pa_ref[0], preferred_element_type=jnp.float32)
        + jnp.dot(y_p[1], pa_ref[1], preferred_element_type=jnp.float32))


def _stage_d(dis_pk, acc2, hcat2, pa, b2p):
    return pl.pallas_call(
        _stage_d_kernel,
        grid=(GRID_R,),
        in_specs=[
            pl.BlockSpec((RP, 256), lambda i: (i, 0)),
            pl.BlockSpec((NC, RP, 256), lambda i: (0, i, 0)),
            pl.BlockSpec((NC, RP, 256), lambda i: (0, i, 0)),
            pl.BlockSpec((NC, 256, 512), lambda i: (0, 0, 0)),
            pl.BlockSpec((NC, 256), lambda i: (0, 0)),
        ],
        out_specs=pl.BlockSpec((RP, 512), lambda i: (i, 0)),
        out_shape=jax.ShapeDtypeStruct((N_NODES * D // 512, 512), jnp.float32),
    )(dis_pk, acc2, hcat2, pa, b2p)


def kernel(z, edge_index, W1, b1, W2, b2):
    src = edge_index[0].astype(jnp.int32)
    dst = edge_index[1].astype(jnp.int32)
    pad = jnp.full((E_PAD - E_EDGES,), N_NODES, jnp.int32)
    src2 = jnp.concatenate([src, pad]).reshape(E_ROWS, LANES)
    dst2 = jnp.concatenate([dst, pad]).reshape(E_ROWS, LANES)

    eye8 = jnp.eye(8, dtype=jnp.float32)
    w1k = jnp.stack([jnp.kron(eye8, W1[:, 32 * c:32 * c + 32])
                     for c in range(NC)])
    w2k = jnp.stack([
        jnp.stack([jnp.kron(eye8, W2[32 * h:32 * h + 32, 32 * c:32 * c + 32])
                   for c in range(NC)])
        for h in range(NC)])
    b1p = jnp.stack([jnp.tile(b1[32 * c:32 * c + 32], 8) for c in range(NC)])
    b2p = jnp.stack([jnp.tile(b2[32 * c:32 * c + 32], 8) for c in range(NC)])
    i32 = np.arange(256)
    j8, k32 = i32 // 32, i32 % 32
    pa_np = np.zeros((NC, 256, 512), np.float32)
    for h in range(NC):
        pa_np[h, i32, 64 * j8 + 32 * h + k32] = 1.0
    pa = jnp.asarray(pa_np)

    z_p = z.reshape(N_NODES * D // 512, 512)
    deg_parts = _sc_degree(dst2).reshape(NC * NS, N_PAD)
    hcat1, dis_pk = _stage_b(deg_parts, z_p, w1k)
    acc1 = _sc_edge_aggregate(src2, dst2, hcat1.reshape(NC, N_PAD, DH))
    hcat2 = _stage_c(dis_pk, acc1.reshape(NC, NPP, 256), hcat1, w2k, b1p)
    acc2 = _sc_edge_aggregate(src2, dst2, hcat2.reshape(NC, N_PAD, DH))
    out = _stage_d(dis_pk, acc2.reshape(NC, NPP, 256), hcat2, pa, b2p)
    return out.reshape(N_NODES, D)


# speedup vs baseline: 1.5213x; 1.0183x over previous
import jax
import jax.numpy as jnp
import numpy as np
from jax import lax
from jax.experimental import pallas as pl
from jax.experimental.pallas import tpu as pltpu
from jax.experimental.pallas import tpu_sc as plsc

N_NODES = 50000
N_PAD = 50176
E_EDGES = 800000
E_ROWS = 6400
E_PAD = E_ROWS * 128
LANES = 128
NC, NS = 2, 16
D = 64
DH = 32
ROWS_PER_TILE_AGG = E_ROWS // NS
ROWS_PER_WORKER_DEG = E_ROWS // (NC * NS)
NODES_PER_TILE = N_PAD // NS
RB = 512
GRID_R = N_PAD // RB
KBUF = 10
CH = 50
N_CHUNKS = ROWS_PER_TILE_AGG // CH
EDT = jnp.bfloat16


def _sc_degree_body(ei_hbm, deg_parts_hbm, dst_v, acc_v):
    c = lax.axis_index("c")
    s = lax.axis_index("s")
    w = c * NS + s
    ones16 = jnp.ones((16,), jnp.float32)
    zer = jnp.zeros((16,), jnp.float32)

    @pl.loop(0, N_PAD // 128)
    def _(i):
        for u in range(8):
            acc_v[pl.ds(i * 128 + u * 16, 16)] = zer

    row0 = w * ROWS_PER_WORKER_DEG
    pltpu.sync_copy(ei_hbm.at[1, pl.ds(row0, ROWS_PER_WORKER_DEG)], dst_v)

    @pl.loop(0, ROWS_PER_WORKER_DEG)
    def _(j):
        for u in range(8):
            idx = dst_v[j, pl.ds(u * 16, 16)]
            plsc.addupdate_scatter(acc_v, [idx], ones16)

    pltpu.sync_copy(acc_v, deg_parts_hbm.at[pl.ds(w * N_PAD, N_PAD)])


def _sc_degree(ei3):
    mesh = plsc.VectorSubcoreMesh(core_axis_name="c", subcore_axis_name="s")
    kern = pl.kernel(
        _sc_degree_body,
        out_type=jax.ShapeDtypeStruct((NC * NS * N_PAD,), jnp.float32),
        compiler_params=pltpu.CompilerParams(needs_layout_passes=False),
        mesh=mesh,
        scratch_types=[
            pltpu.VMEM((ROWS_PER_WORKER_DEG, LANES), jnp.int32),
            pltpu.VMEM((N_PAD,), jnp.float32),
        ],
    )
    return kern(ei3)


def _sc_agg_body(ei_hbm, table_hbm, acc_hbm,
                 src_v, dst_v, rows_v, acc_sh, sem_g, sem_s):
    c = lax.axis_index("c")
    s = lax.axis_index("s")

    zer = jnp.zeros((DH,), EDT)

    @pl.loop(0, LANES)
    def _(j):
        rows_v[0, j, :] = zer

    node0 = s * NODES_PER_TILE
    nfull = NODES_PER_TILE // 128

    @pl.loop(0, nfull)
    def _(i):
        pltpu.sync_copy(rows_v.at[0],
                        acc_sh.at[pl.ds(node0 + i * 128, 128)])

    pltpu.sync_copy(rows_v.at[0, pl.ds(0, NODES_PER_TILE - nfull * 128)],
                    acc_sh.at[pl.ds(node0 + nfull * 128,
                                    NODES_PER_TILE - nfull * 128)])

    plsc.subcore_barrier()

    row0 = s * ROWS_PER_TILE_AGG
    table_c = table_hbm.at[c]

    def g_start(j, b):
        pltpu.async_copy(table_c.at[src_v.at[j]], rows_v.at[b], sem_g.at[b])

    def g_wait(j, b):
        pltpu.make_async_copy(table_c.at[src_v.at[j]], rows_v.at[b],
                              sem_g.at[b]).wait()

    def s_start(j, b):
        pltpu.async_copy(rows_v.at[b], acc_sh.at[dst_v.at[j]], sem_s.at[b],
                         add=True)

    def s_wait(j, b):
        pltpu.make_async_copy(rows_v.at[b], acc_sh.at[dst_v.at[j]],
                              sem_s.at[b]).wait()

    @pl.loop(0, N_CHUNKS)
    def _(ch):
        r0 = row0 + ch * CH
        pltpu.sync_copy(ei_hbm.at[0, pl.ds(r0, CH)], src_v)
        pltpu.sync_copy(ei_hbm.at[1, pl.ds(r0, CH)], dst_v)

        for b in range(KBUF):
            g_start(b, b)

        @pl.loop(0, CH // KBUF - 1)
        def _(g):
            r = g * KBUF
            for b in range(KBUF):
                g_wait(r + b, b)
                s_start(r + b, b)
            for b in range(KBUF):
                s_wait(r + b, b)
                g_start(r + KBUF + b, b)

        r_last = CH - KBUF
        for b in range(KBUF):
            g_wait(r_last + b, b)
            s_start(r_last + b, b)
        for b in range(KBUF):
            s_wait(r_last + b, b)

    plsc.subcore_barrier()

    pltpu.sync_copy(acc_sh.at[pl.ds(node0, NODES_PER_TILE)],
                    acc_hbm.at[c, pl.ds(node0, NODES_PER_TILE)])


def _sc_edge_aggregate(ei3, table):
    mesh = plsc.VectorSubcoreMesh(core_axis_name="c", subcore_axis_name="s")
    kern = pl.kernel(
        _sc_agg_body,
        out_type=jax.ShapeDtypeStruct((NC, N_PAD, DH), EDT),
        compiler_params=pltpu.CompilerParams(use_tc_tiling_on_sc=False),
        mesh=mesh,
        scratch_types=[
            pltpu.VMEM((CH, LANES), jnp.int32),
            pltpu.VMEM((CH, LANES), jnp.int32),
            pltpu.VMEM((KBUF, LANES, DH), EDT),
            pltpu.VMEM_SHARED((N_PAD, DH), EDT),
            pltpu.SemaphoreType.DMA((KBUF,)),
            pltpu.SemaphoreType.DMA((KBUF,)),
        ],
    )
    return kern(ei3, table)


def _dis_block(parts_ref):
    ones = jnp.ones((NC * NS, 1), jnp.float32)
    deg = lax.dot_general(parts_ref[...], ones, (((0,), (0,)), ((), ())),
                          preferred_element_type=jnp.float32) + 1.0
    return lax.rsqrt(deg)


RP = RB // 8
NPP = N_PAD // 8


def _pack_mats(w):
    q = lax.broadcasted_iota(jnp.int32, (RP, RB), 0)
    r0 = lax.broadcasted_iota(jnp.int32, (RP, RB), 1)
    t2 = (r0 // 8 == q).astype(jnp.float32)
    r1 = lax.broadcasted_iota(jnp.int32, (RB, 8 * w), 0)
    l1 = lax.broadcasted_iota(jnp.int32, (RB, 8 * w), 1)
    m = (r1 % 8 == l1 // w).astype(jnp.float32)
    return t2, m


def _pack8(x, w):
    t2, m = _pack_mats(w)
    tiled = jnp.concatenate([x] * 8, axis=1)
    return lax.dot_general(t2, tiled * m, (((1,), (0,)), ((), ())),
                           preferred_element_type=jnp.float32)


def _dis_packed(parts_ref):
    dis = _dis_block(parts_ref)
    return _pack8(jnp.broadcast_to(dis, (RB, DH)), DH)


def _stage_b_kernel(parts_ref, zp_ref, w1k_ref, hcat_ref, disp_ref):
    dis_p = _dis_packed(parts_ref)
    for c in range(NC):
        hp = jnp.dot(zp_ref[...], w1k_ref[c],
                     preferred_element_type=jnp.float32)
        hcat_ref[c] = (hp * dis_p).astype(EDT)
    disp_ref[...] = dis_p


def _stage_b(deg_parts, z_p, w1k):
    return pl.pallas_call(
        _stage_b_kernel,
        grid=(GRID_R,),
        in_specs=[
            pl.BlockSpec((NC * NS, RB), lambda i: (0, i)),
            pl.BlockSpec((RP, 512), lambda i: (i, 0)),
            pl.BlockSpec((NC, 512, 256), lambda i: (0, 0, 0)),
        ],
        out_specs=[
            pl.BlockSpec((NC, RP, 256), lambda i: (0, i, 0)),
            pl.BlockSpec((RP, 256), lambda i: (i, 0)),
        ],
        out_shape=[
            jax.ShapeDtypeStruct((NC, NPP, 256), EDT),
            jax.ShapeDtypeStruct((NPP, 256), jnp.float32),
        ],
    )(deg_parts, z_p, w1k)


def _stage_c_kernel(disp_ref, acc_ref, hcat_ref, w2k_ref, b1_ref, hcat2_ref):
    dis_p = disp_ref[...]
    x_p = []
    for c in range(NC):
        a = acc_ref[c].astype(jnp.float32)
        h = hcat_ref[c].astype(jnp.float32)
        x_p.append(jnp.maximum((a + h) * dis_p + b1_ref[c], 0.0))
    for c in range(NC):
        h2 = (jnp.dot(x_p[0], w2k_ref[0, c],
                      preferred_element_type=jnp.float32)
              + jnp.dot(x_p[1], w2k_ref[1, c],
                        preferred_element_type=jnp.float32))
        hcat2_ref[c] = (h2 * dis_p).astype(EDT)


def _stage_c(dis_pk, acc1, hcat1, w2k, b1p):
    return pl.pallas_call(
        _stage_c_kernel,
        grid=(GRID_R,),
        in_specs=[
            pl.BlockSpec((RP, 256), lambda i: (i, 0)),
            pl.BlockSpec((NC, RP, 256), lambda i: (0, i, 0)),
            pl.BlockSpec((NC, RP, 256), lambda i: (0, i, 0)),
            pl.BlockSpec((NC, NC, 256, 256), lambda i: (0, 0, 0, 0)),
            pl.BlockSpec((NC, 256), lambda i: (0, 0)),
        ],
        out_specs=pl.BlockSpec((NC, RP, 256), lambda i: (0, i, 0)),
        out_shape=jax.ShapeDtypeStruct((NC, NPP, 256), EDT),
    )(dis_pk, acc1, hcat1, w2k, b1p)


def _stage_d_kernel(disp_ref, acc_ref, hcat_ref, pa_ref, b2_ref, out_ref):
    dis_p = disp_ref[...]
    y_p = []
    for c in range(NC):
        a = acc_ref[c].astype(jnp.float32)
        h = hcat_ref[c].astype(jnp.float32)
        y_p.append(jnp.maximum((a + h) * dis_p + b2_ref[c], 0.0))
    out_ref[...] = (
        jnp.dot(y_p[0], pa_ref[0], preferred_element_type=jnp.float32)
        + jnp.dot(y_p[1], pa_ref[1], preferred_element_type=jnp.float32))


def _stage_d(dis_pk, acc2, hcat2, pa, b2p):
    return pl.pallas_call(
        _stage_d_kernel,
        grid=(GRID_R,),
        in_specs=[
            pl.BlockSpec((RP, 256), lambda i: (i, 0)),
            pl.BlockSpec((NC, RP, 256), lambda i: (0, i, 0)),
            pl.BlockSpec((NC, RP, 256), lambda i: (0, i, 0)),
            pl.BlockSpec((NC, 256, 512), lambda i: (0, 0, 0)),
            pl.BlockSpec((NC, 256), lambda i: (0, 0)),
        ],
        out_specs=pl.BlockSpec((RP, 512), lambda i: (i, 0)),
        out_shape=jax.ShapeDtypeStruct((N_NODES * D // 512, 512), jnp.float32),
    )(dis_pk, acc2, hcat2, pa, b2p)


def kernel(z, edge_index, W1, b1, W2, b2):
    pad = jnp.full((2, E_PAD - E_EDGES), N_NODES, jnp.int32)
    ei3 = jnp.concatenate([edge_index.astype(jnp.int32), pad],
                          axis=1).reshape(2, E_ROWS, LANES)

    eye8 = jnp.eye(8, dtype=jnp.float32)
    w1k = jnp.stack([jnp.kron(eye8, W1[:, 32 * c:32 * c + 32])
                     for c in range(NC)])
    w2k = jnp.stack([
        jnp.stack([jnp.kron(eye8, W2[32 * h:32 * h + 32, 32 * c:32 * c + 32])
                   for c in range(NC)])
        for h in range(NC)])
    b1p = jnp.stack([jnp.tile(b1[32 * c:32 * c + 32], 8) for c in range(NC)])
    b2p = jnp.stack([jnp.tile(b2[32 * c:32 * c + 32], 8) for c in range(NC)])
    i32 = np.arange(256)
    j8, k32 = i32 // 32, i32 % 32
    pa_np = np.zeros((NC, 256, 512), np.float32)
    for h in range(NC):
        pa_np[h, i32, 64 * j8 + 32 * h + k32] = 1.0
    pa = jnp.asarray(pa_np)

    z_p = z.reshape(N_NODES * D // 512, 512)
    deg_parts = _sc_degree(ei3).reshape(NC * NS, N_PAD)
    hcat1, dis_pk = _stage_b(deg_parts, z_p, w1k)
    acc1 = _sc_edge_aggregate(ei3, hcat1.reshape(NC, N_PAD, DH))
    hcat2 = _stage_c(dis_pk, acc1.reshape(NC, NPP, 256), hcat1, w2k, b1p)
    acc2 = _sc_edge_aggregate(ei3, hcat2.reshape(NC, N_PAD, DH))
    out = _stage_d(dis_pk, acc2.reshape(NC, NPP, 256), hcat2, pa, b2p)
    return out.reshape(N_NODES, D)


# speedup vs baseline: 1.5228x; 1.0010x over previous
import jax
import jax.numpy as jnp
import numpy as np
from jax import lax
from jax.experimental import pallas as pl
from jax.experimental.pallas import tpu as pltpu
from jax.experimental.pallas import tpu_sc as plsc

N_NODES = 50000
N_PAD = 50176
E_EDGES = 800000
E_ROWS = 6400
E_PAD = E_ROWS * 128
LANES = 128
NC, NS = 2, 16
D = 64
DH = 32
ROWS_PER_TILE_AGG = E_ROWS // NS
ROWS_PER_WORKER_DEG = E_ROWS // (NC * NS)
NODES_PER_TILE = N_PAD // NS
RB = 512
GRID_R = N_PAD // RB
KBUF = 10
CH = 50
N_CHUNKS = ROWS_PER_TILE_AGG // CH
EDT = jnp.bfloat16


def _sc_degree_body(ei_hbm, deg_parts_hbm, dst_v, acc_v):
    c = lax.axis_index("c")
    s = lax.axis_index("s")
    w = c * NS + s
    ones16 = jnp.ones((16,), jnp.float32)
    zer = jnp.zeros((16,), jnp.float32)

    @pl.loop(0, N_PAD // 128)
    def _(i):
        for u in range(8):
            acc_v[pl.ds(i * 128 + u * 16, 16)] = zer

    row0 = w * ROWS_PER_WORKER_DEG
    pltpu.sync_copy(ei_hbm.at[1, pl.ds(row0, ROWS_PER_WORKER_DEG)], dst_v)

    @pl.loop(0, ROWS_PER_WORKER_DEG)
    def _(j):
        for u in range(8):
            idx = dst_v[j, pl.ds(u * 16, 16)]
            plsc.addupdate_scatter(acc_v, [idx], ones16)

    pltpu.sync_copy(acc_v, deg_parts_hbm.at[pl.ds(w * N_PAD, N_PAD)])


def _sc_degree(ei3):
    mesh = plsc.VectorSubcoreMesh(core_axis_name="c", subcore_axis_name="s")
    kern = pl.kernel(
        _sc_degree_body,
        out_type=jax.ShapeDtypeStruct((NC * NS * N_PAD,), jnp.float32),
        compiler_params=pltpu.CompilerParams(needs_layout_passes=False),
        mesh=mesh,
        scratch_types=[
            pltpu.VMEM((ROWS_PER_WORKER_DEG, LANES), jnp.int32),
            pltpu.VMEM((N_PAD,), jnp.float32),
        ],
    )
    return kern(ei3)


def _sc_agg_body(ei_hbm, table_hbm, acc_hbm,
                 src_v, dst_v, rows_v, acc_sh, sem_g, sem_s):
    c = lax.axis_index("c")
    s = lax.axis_index("s")

    zer = jnp.zeros((DH,), EDT)

    @pl.loop(0, LANES)
    def _(j):
        rows_v[0, j, :] = zer

    node0 = s * NODES_PER_TILE
    nfull = NODES_PER_TILE // 128

    @pl.loop(0, nfull)
    def _(i):
        pltpu.sync_copy(rows_v.at[0],
                        acc_sh.at[pl.ds(node0 + i * 128, 128)])

    pltpu.sync_copy(rows_v.at[0, pl.ds(0, NODES_PER_TILE - nfull * 128)],
                    acc_sh.at[pl.ds(node0 + nfull * 128,
                                    NODES_PER_TILE - nfull * 128)])

    plsc.subcore_barrier()

    row0 = s * ROWS_PER_TILE_AGG
    table_c = table_hbm.at[c]

    def g_start(j, b):
        pltpu.async_copy(table_c.at[src_v.at[j]], rows_v.at[b], sem_g.at[b])

    def g_wait(j, b):
        pltpu.make_async_copy(table_c.at[src_v.at[j]], rows_v.at[b],
                              sem_g.at[b]).wait()

    def s_start(j, b):
        pltpu.async_copy(rows_v.at[b], acc_sh.at[dst_v.at[j]], sem_s.at[b],
                         add=True)

    def s_wait(j, b):
        pltpu.make_async_copy(rows_v.at[b], acc_sh.at[dst_v.at[j]],
                              sem_s.at[b]).wait()

    @pl.loop(0, N_CHUNKS)
    def _(ch):
        r0 = row0 + ch * CH
        pltpu.sync_copy(ei_hbm.at[0, pl.ds(r0, CH)], src_v)
        pltpu.sync_copy(ei_hbm.at[1, pl.ds(r0, CH)], dst_v)

        for b in range(KBUF):
            g_start(b, b)

        @pl.loop(0, CH // KBUF - 1)
        def _(g):
            r = g * KBUF
            for b in range(KBUF):
                g_wait(r + b, b)
                s_start(r + b, b)
            for b in range(KBUF):
                s_wait(r + b, b)
                g_start(r + KBUF + b, b)

        r_last = CH - KBUF
        for b in range(KBUF):
            g_wait(r_last + b, b)
            s_start(r_last + b, b)
        for b in range(KBUF):
            s_wait(r_last + b, b)

    plsc.subcore_barrier()

    pltpu.sync_copy(acc_sh.at[pl.ds(node0, NODES_PER_TILE)],
                    acc_hbm.at[c, pl.ds(node0, NODES_PER_TILE)])


def _sc_edge_aggregate(ei3, table):
    mesh = plsc.VectorSubcoreMesh(core_axis_name="c", subcore_axis_name="s")
    kern = pl.kernel(
        _sc_agg_body,
        out_type=jax.ShapeDtypeStruct((NC, N_PAD, DH), EDT),
        compiler_params=pltpu.CompilerParams(use_tc_tiling_on_sc=False),
        mesh=mesh,
        scratch_types=[
            pltpu.VMEM((CH, LANES), jnp.int32),
            pltpu.VMEM((CH, LANES), jnp.int32),
            pltpu.VMEM((KBUF, LANES, DH), EDT),
            pltpu.VMEM_SHARED((N_PAD, DH), EDT),
            pltpu.SemaphoreType.DMA((KBUF,)),
            pltpu.SemaphoreType.DMA((KBUF,)),
        ],
    )
    return kern(ei3, table)


def _dis_block(parts_ref):
    ones = jnp.ones((NC * NS, 1), jnp.float32)
    deg = lax.dot_general(parts_ref[...], ones, (((0,), (0,)), ((), ())),
                          preferred_element_type=jnp.float32) + 1.0
    return lax.rsqrt(deg)


RP = RB // 8
NPP = N_PAD // 8


def _pack_mats(w):
    q = lax.broadcasted_iota(jnp.int32, (RP, RB), 0)
    r0 = lax.broadcasted_iota(jnp.int32, (RP, RB), 1)
    t2 = (r0 // 8 == q).astype(jnp.float32)
    r1 = lax.broadcasted_iota(jnp.int32, (RB, 8 * w), 0)
    l1 = lax.broadcasted_iota(jnp.int32, (RB, 8 * w), 1)
    m = (r1 % 8 == l1 // w).astype(jnp.float32)
    return t2, m


def _pack8(x, w):
    t2, m = _pack_mats(w)
    tiled = jnp.concatenate([x] * 8, axis=1)
    return lax.dot_general(t2, tiled * m, (((1,), (0,)), ((), ())),
                           preferred_element_type=jnp.float32)


def _dis_packed(parts_ref):
    dis = _dis_block(parts_ref)
    return _pack8(jnp.broadcast_to(dis, (RB, DH)), DH)


def _stage_b_kernel(parts_ref, zp_ref, w1k_ref, hcat_ref, disp_ref):
    dis_p = _dis_packed(parts_ref)
    for c in range(NC):
        hp = jnp.dot(zp_ref[...].astype(EDT), w1k_ref[c],
                     preferred_element_type=jnp.float32)
        hcat_ref[c] = (hp * dis_p).astype(EDT)
    disp_ref[...] = dis_p


def _stage_b(deg_parts, z_p, w1k):
    return pl.pallas_call(
        _stage_b_kernel,
        grid=(GRID_R,),
        in_specs=[
            pl.BlockSpec((NC * NS, RB), lambda i: (0, i)),
            pl.BlockSpec((RP, 512), lambda i: (i, 0)),
            pl.BlockSpec((NC, 512, 256), lambda i: (0, 0, 0)),
        ],
        out_specs=[
            pl.BlockSpec((NC, RP, 256), lambda i: (0, i, 0)),
            pl.BlockSpec((RP, 256), lambda i: (i, 0)),
        ],
        out_shape=[
            jax.ShapeDtypeStruct((NC, NPP, 256), EDT),
            jax.ShapeDtypeStruct((NPP, 256), jnp.float32),
        ],
    )(deg_parts, z_p, w1k)


def _stage_c_kernel(disp_ref, acc_ref, hcat_ref, w2k_ref, b1_ref, hcat2_ref):
    dis_p = disp_ref[...]
    x_p = []
    for c in range(NC):
        a = acc_ref[c].astype(jnp.float32)
        h = hcat_ref[c].astype(jnp.float32)
        x_p.append(jnp.maximum((a + h) * dis_p + b1_ref[c], 0.0))
    for c in range(NC):
        h2 = (jnp.dot(x_p[0].astype(EDT), w2k_ref[0, c],
                      preferred_element_type=jnp.float32)
              + jnp.dot(x_p[1].astype(EDT), w2k_ref[1, c],
                        preferred_element_type=jnp.float32))
        hcat2_ref[c] = (h2 * dis_p).astype(EDT)


def _stage_c(dis_pk, acc1, hcat1, w2k, b1p):
    return pl.pallas_call(
        _stage_c_kernel,
        grid=(GRID_R,),
        in_specs=[
            pl.BlockSpec((RP, 256), lambda i: (i, 0)),
            pl.BlockSpec((NC, RP, 256), lambda i: (0, i, 0)),
            pl.BlockSpec((NC, RP, 256), lambda i: (0, i, 0)),
            pl.BlockSpec((NC, NC, 256, 256), lambda i: (0, 0, 0, 0)),
            pl.BlockSpec((NC, 256), lambda i: (0, 0)),
        ],
        out_specs=pl.BlockSpec((NC, RP, 256), lambda i: (0, i, 0)),
        out_shape=jax.ShapeDtypeStruct((NC, NPP, 256), EDT),
    )(dis_pk, acc1, hcat1, w2k, b1p)


def _stage_d_kernel(disp_ref, acc_ref, hcat_ref, pa_ref, b2_ref, out_ref):
    dis_p = disp_ref[...]
    y_p = []
    for c in range(NC):
        a = acc_ref[c].astype(jnp.float32)
        h = hcat_ref[c].astype(jnp.float32)
        y_p.append(jnp.maximum((a + h) * dis_p + b2_ref[c], 0.0))
    out_ref[...] = (
        jnp.dot(y_p[0].astype(EDT), pa_ref[0],
                preferred_element_type=jnp.float32)
        + jnp.dot(y_p[1].astype(EDT), pa_ref[1],
                  preferred_element_type=jnp.float32))


def _stage_d(dis_pk, acc2, hcat2, pa, b2p):
    return pl.pallas_call(
        _stage_d_kernel,
        grid=(GRID_R,),
        in_specs=[
            pl.BlockSpec((RP, 256), lambda i: (i, 0)),
            pl.BlockSpec((NC, RP, 256), lambda i: (0, i, 0)),
            pl.BlockSpec((NC, RP, 256), lambda i: (0, i, 0)),
            pl.BlockSpec((NC, 256, 512), lambda i: (0, 0, 0)),
            pl.BlockSpec((NC, 256), lambda i: (0, 0)),
        ],
        out_specs=pl.BlockSpec((RP, 512), lambda i: (i, 0)),
        out_shape=jax.ShapeDtypeStruct((N_NODES * D // 512, 512), jnp.float32),
    )(dis_pk, acc2, hcat2, pa, b2p)


def kernel(z, edge_index, W1, b1, W2, b2):
    pad = jnp.full((2, E_PAD - E_EDGES), N_NODES, jnp.int32)
    ei3 = jnp.concatenate([edge_index.astype(jnp.int32), pad],
                          axis=1).reshape(2, E_ROWS, LANES)

    eye8 = jnp.eye(8, dtype=jnp.float32)
    w1k = jnp.stack([jnp.kron(eye8, W1[:, 32 * c:32 * c + 32])
                     for c in range(NC)]).astype(EDT)
    w2k = jnp.stack([
        jnp.stack([jnp.kron(eye8, W2[32 * h:32 * h + 32, 32 * c:32 * c + 32])
                   for c in range(NC)])
        for h in range(NC)]).astype(EDT)
    b1p = jnp.stack([jnp.tile(b1[32 * c:32 * c + 32], 8) for c in range(NC)])
    b2p = jnp.stack([jnp.tile(b2[32 * c:32 * c + 32], 8) for c in range(NC)])
    i32 = np.arange(256)
    j8, k32 = i32 // 32, i32 % 32
    pa_np = np.zeros((NC, 256, 512), np.float32)
    for h in range(NC):
        pa_np[h, i32, 64 * j8 + 32 * h + k32] = 1.0
    pa = jnp.asarray(pa_np).astype(EDT)

    z_p = z.reshape(N_NODES * D // 512, 512)
    deg_parts = _sc_degree(ei3).reshape(NC * NS, N_PAD)
    hcat1, dis_pk = _stage_b(deg_parts, z_p, w1k)
    acc1 = _sc_edge_aggregate(ei3, hcat1.reshape(NC, N_PAD, DH))
    hcat2 = _stage_c(dis_pk, acc1.reshape(NC, NPP, 256), hcat1, w2k, b1p)
    acc2 = _sc_edge_aggregate(ei3, hcat2.reshape(NC, N_PAD, DH))
    out = _stage_d(dis_pk, acc2.reshape(NC, NPP, 256), hcat2, pa, b2p)
    return out.reshape(N_NODES, D)
